# baseline (speedup 1.0000x reference)
import functools

import jax
import jax.numpy as jnp
from jax import lax
from jax.experimental import pallas as pl
from jax.experimental.pallas import tpu as pltpu
from jax.experimental.pallas import tpu_sc as plsc

VOCAB = 1000000
DIM = 64
B = 4096
L = 200

NC = 2
NS = 16
NW = NC * NS
ROWS_W = B // NW
SPLIT = (104, 96)
OFFS = (0, 104)
NCH = ROWS_W * 2
NBUF = 4
LOOK = 2


@functools.partial(
    pl.kernel,
    out_type=jax.ShapeDtypeStruct((B, L, DIM), jnp.float32),
    mesh=plsc.VectorSubcoreMesh(core_axis_name="c", subcore_axis_name="s"),
    compiler_params=pltpu.CompilerParams(use_tc_tiling_on_sc=False),
    scratch_types=[
        pltpu.VMEM((ROWS_W, L), jnp.int32),
        pltpu.VMEM((NBUF, 128, DIM), jnp.float32),
    ] + [pltpu.SemaphoreType.DMA] * (2 * NBUF),
)
def _emb_lookup(idx_hbm, table_hbm, out_hbm, idx_v, rows_v, *sems):
    gsem = sems[:NBUF]
    wsem = sems[NBUF:]
    wid = lax.axis_index("s") * NC + lax.axis_index("c")
    row0 = wid * ROWS_W

    pltpu.sync_copy(idx_hbm.at[pl.ds(row0, ROWS_W)], idx_v)

    def gather_copy(i, p, b):
        return pltpu.make_async_copy(
            table_hbm.at[idx_v.at[i, pl.ds(OFFS[p], SPLIT[p])]],
            rows_v.at[b, pl.ds(0, SPLIT[p])],
            gsem[b])

    def write_copy(i, p, b):
        return pltpu.make_async_copy(
            rows_v.at[b, pl.ds(0, SPLIT[p])],
            out_hbm.at[row0 + i, pl.ds(OFFS[p], SPLIT[p])],
            wsem[b])

    for t in range(LOOK):
        gather_copy(t // 2, t % 2, t % NBUF).start()
    for t in range(LOOK):
        i, p, b = t // 2, t % 2, t % NBUF
        t2 = t + LOOK
        gather_copy(i, p, b).wait()
        write_copy(i, p, b).start()
        gather_copy(t2 // 2, t2 % 2, t2 % NBUF).start()

    n_steady = NCH - 2 * LOOK

    def outer(jj, _):
        for u in range(NBUF):
            p = (LOOK + u) % 2
            b = (LOOK + u) % NBUF
            b2 = u % NBUF
            i = 2 * jj + (LOOK + u) // 2
            i_prev = i - 1
            i_next = i + 1
            gather_copy(i, p, b).wait()
            write_copy(i, p, b).start()
            write_copy(i_prev, p, b2).wait()
            gather_copy(i_next, p, b2).start()
        return 0

    lax.fori_loop(0, n_steady // NBUF, outer, 0)

    for t in range(NCH - LOOK, NCH):
        i, p, b = t // 2, t % 2, t % NBUF
        b2 = (t + LOOK) % NBUF
        gather_copy(i, p, b).wait()
        write_copy(i, p, b).start()
        write_copy(i - 1, p, b2).wait()

    for t in range(NCH - LOOK, NCH):
        write_copy(t // 2, t % 2, t % NBUF).wait()


def kernel(token_ids, table):
    return _emb_lookup(token_ids, table)


# speedup vs baseline: 1.3408x; 1.3408x over previous
import functools

import jax
import jax.numpy as jnp
from jax import lax
from jax.experimental import pallas as pl
from jax.experimental.pallas import tpu as pltpu
from jax.experimental.pallas import tpu_sc as plsc

VOCAB = 1000000
DIM = 64
B = 4096
L = 200

NC = 2
NS = 16
NW = NC * NS
ROWS_W = B // NW
TOK_W = ROWS_W * L
NBUF = 2
NG = L // 16
REM = L % 16


@functools.partial(
    pl.kernel,
    out_type=jax.ShapeDtypeStruct((B, L, DIM), jnp.float32),
    mesh=plsc.VectorSubcoreMesh(core_axis_name="c", subcore_axis_name="s"),
    compiler_params=pltpu.CompilerParams(use_tc_tiling_on_sc=True),
    scratch_types=[
        pltpu.VMEM((TOK_W + 16,), jnp.int32),
        pltpu.VMEM((NBUF, L, DIM), jnp.float32),
        pltpu.SemaphoreType.DMA,
        pltpu.SemaphoreType.DMA,
        pltpu.SemaphoreType.DMA,
        pltpu.SemaphoreType.DMA,
    ],
)
def _emb_lookup(idx_hbm, table_hbm, out_hbm, idx_v, rows_v, g0, g1, w0, w1):
    gsem = (g0, g1)
    wsem = (w0, w1)
    wid = lax.axis_index("s") * NC + lax.axis_index("c")
    row0 = wid * ROWS_W
    base = row0 * L

    pltpu.sync_copy(idx_hbm.at[pl.ds(base, TOK_W)], idx_v.at[pl.ds(0, TOK_W)])

    def enqueue_row(i, b):
        def group(g, _):
            pos = i * L + g * 16
            v = idx_v[pl.ds(pos, 16)]
            for j in range(16):
                pltpu.make_async_copy(
                    table_hbm.at[v[j]],
                    rows_v.at[b, g * 16 + j],
                    gsem[b]).start()
            return 0
        lax.fori_loop(0, NG, group, 0)
        v = idx_v[pl.ds(i * L + NG * 16, 16)]
        for j in range(REM):
            pltpu.make_async_copy(
                table_hbm.at[v[j]],
                rows_v.at[b, NG * 16 + j],
                gsem[b]).start()

    def drain_row(b):
        pltpu.make_async_copy(
            table_hbm.at[pl.ds(0, L)], rows_v.at[b], gsem[b]).wait()

    def write_copy(i, b):
        return pltpu.make_async_copy(
            rows_v.at[b], out_hbm.at[row0 + i], wsem[b])

    def step(i2, _):
        for b in range(NBUF):
            i = i2 * NBUF + b

            @pl.when(i >= 2)
            def _():
                write_copy(i - 2, b).wait()

            enqueue_row(i, b)

            @pl.when(i >= 1)
            def _():
                drain_row(1 - b)
                write_copy(i - 1, 1 - b).start()
        return 0

    lax.fori_loop(0, ROWS_W // NBUF, step, 0)

    drain_row(1)
    write_copy(ROWS_W - 1, 1).start()
    write_copy(ROWS_W - 2, 0).wait()
    write_copy(ROWS_W - 1, 1).wait()


def kernel(token_ids, table):
    return _emb_lookup(token_ids.reshape(-1), table)
